# baseline (speedup 1.0000x reference)
import functools

import jax
import jax.numpy as jnp
from jax import lax
from jax.experimental import pallas as pl
from jax.experimental.pallas import tpu as pltpu
from jax.experimental.pallas import tpu_sc as plsc

N = 10000
E = 320000
D = 128

NC = 2
NS = 16
NW = NC * NS
CHUNK = 128
CPT0 = 160
CPT1 = 0
NCHP = NS * (CPT0 + CPT1)
EPAD = NCHP * CHUNK - E
NP = 10240
RPT = NP // NS
GJ = 16
L = 16


def _sc_body(src_hbm, dst_hbm, h_hbm, zagg_hbm, zdeg_hbm,
             agg_out, deg_out,
             agg_sh, srcg, dstg, rows0, rows1, deg_local,
             sem0, sem1):
    c = lax.axis_index("c")
    s = lax.axis_index("s")
    wid = c * NS + s
    start = jnp.where(c == 0, s * CPT0, NS * CPT0 + s * CPT1)
    ngroups = jnp.where(c == 0, CPT0 // GJ, CPT1 // GJ)
    row0 = s * RPT

    pltpu.sync_copy(zagg_hbm, rows0)
    pltpu.sync_copy(zagg_hbm, rows1)
    for r in range(RPT // CHUNK):
        buf = rows0 if r % 2 == 0 else rows1
        pltpu.sync_copy(buf, agg_sh.at[pl.ds(row0 + r * CHUNK, CHUNK)])
    pltpu.sync_copy(zdeg_hbm, deg_local)

    plsc.subcore_barrier()

    ones = jnp.ones((L,), jnp.float32)

    def _deg_add(jrow):
        for k in range(CHUNK // L):
            idx = dstg[jrow, pl.ds(k * L, L)]
            plsc.addupdate_scatter(deg_local, [idx], ones)

    def _group(g, _):
        pltpu.sync_copy(src_hbm.at[pl.ds(start + g * GJ, GJ)], srcg)
        pltpu.sync_copy(dst_hbm.at[pl.ds(start + g * GJ, GJ)], dstg)

        def _pair(p, _):
            j0 = 2 * p
            j1 = j0 + 1
            d0 = pltpu.async_copy(h_hbm.at[srcg.at[j0]], rows0, sem0)
            d1 = pltpu.async_copy(h_hbm.at[srcg.at[j1]], rows1, sem1)
            d0.wait()
            pltpu.sync_copy(rows0, agg_sh.at[dstg.at[j0]], add=True)
            _deg_add(j0)
            d1.wait()
            pltpu.sync_copy(rows1, agg_sh.at[dstg.at[j1]], add=True)
            _deg_add(j1)
            return 0

        return lax.fori_loop(0, GJ // 2, _pair, 0)

    lax.fori_loop(0, ngroups, _group, 0)

    plsc.subcore_barrier()

    for r in range(RPT // CHUNK):
        buf = rows0 if r % 2 == 0 else rows1
        pltpu.sync_copy(agg_sh.at[pl.ds(row0 + r * CHUNK, CHUNK)], buf)
        pltpu.sync_copy(buf, agg_out.at[c, pl.ds(row0 + r * CHUNK, CHUNK)])
    pltpu.sync_copy(deg_local, deg_out.at[wid])


@functools.cache
def _sc_agg():
  return pl.kernel(
    _sc_body,
    out_type=(
        jax.ShapeDtypeStruct((NC, NP, D), jnp.float32),
        jax.ShapeDtypeStruct((NW, NP), jnp.float32),
    ),
    mesh=plsc.VectorSubcoreMesh(core_axis_name="c", subcore_axis_name="s",
                                num_cores=NC, num_subcores=NS),
    compiler_params=pltpu.CompilerParams(needs_layout_passes=False),
    scratch_types=[
        pltpu.VMEM_SHARED((NP, D), jnp.float32),
        pltpu.VMEM((GJ, CHUNK), jnp.int32),
        pltpu.VMEM((GJ, CHUNK), jnp.int32),
        pltpu.VMEM((CHUNK, D), jnp.float32),
        pltpu.VMEM((CHUNK, D), jnp.float32),
        pltpu.VMEM((NP,), jnp.float32),
        pltpu.SemaphoreType.DMA,
        pltpu.SemaphoreType.DMA,
    ],
  )


def _tc_body(h_ref, agg_ref, deg_ref, ws_ref, wn_ref, b_ref, o_ref):
    h = h_ref[...]
    agg = agg_ref[0] + agg_ref[1]
    deg = jnp.sum(deg_ref[...], axis=0)[:, None]
    hn = agg / jnp.maximum(deg, 1.0)
    acc = jnp.dot(h, ws_ref[...], preferred_element_type=jnp.float32)
    acc = acc + jnp.dot(hn, wn_ref[...], preferred_element_type=jnp.float32)
    acc = acc + b_ref[...]
    o_ref[...] = h + jnp.maximum(acc, 0.0)


_TC_ROWS = 1024


def _tc_combine(h_pad, agg2, deg32, W_self, W_neigh, b2d):
    grid = (NP // _TC_ROWS,)
    return pl.pallas_call(
        _tc_body,
        grid=grid,
        in_specs=[
            pl.BlockSpec((_TC_ROWS, D), lambda i: (i, 0)),
            pl.BlockSpec((NC, _TC_ROWS, D), lambda i: (0, i, 0)),
            pl.BlockSpec((NW, _TC_ROWS), lambda i: (0, i)),
            pl.BlockSpec((D, D), lambda i: (0, 0)),
            pl.BlockSpec((D, D), lambda i: (0, 0)),
            pl.BlockSpec((1, D), lambda i: (0, 0)),
        ],
        out_specs=pl.BlockSpec((_TC_ROWS, D), lambda i: (i, 0)),
        out_shape=jax.ShapeDtypeStruct((NP, D), jnp.float32),
    )(h_pad, agg2, deg32, W_self, W_neigh, b2d)


@jax.jit
def kernel(h, edge_index, W_self, W_neigh, b):
    src = edge_index[0].astype(jnp.int32)
    dst = edge_index[1].astype(jnp.int32)
    src2d = jnp.concatenate(
        [src, jnp.zeros((EPAD,), jnp.int32)]).reshape(NCHP, CHUNK)
    dst2d = jnp.concatenate(
        [dst, jnp.full((EPAD,), N, jnp.int32)]).reshape(NCHP, CHUNK)
    zagg = jnp.zeros((CHUNK, D), jnp.float32)
    zdeg = jnp.zeros((NP,), jnp.float32)
    agg2, deg32 = _sc_agg()(src2d, dst2d, h, zagg, zdeg)
    h_pad = jnp.concatenate([h, jnp.zeros((NP - N, D), jnp.float32)])
    out = _tc_combine(h_pad, agg2, deg32, W_self, W_neigh, b.reshape(1, D))
    return out[:N]


# speedup vs baseline: 3.8675x; 3.8675x over previous
import functools

import jax
import jax.numpy as jnp
from jax import lax
from jax.experimental import pallas as pl
from jax.experimental.pallas import tpu as pltpu
from jax.experimental.pallas import tpu_sc as plsc

N = 10000
E = 320000
D = 128

NC = 2
NS = 16
NW = NC * NS
CHUNK = 128
CPT0 = 80
CPT1 = 80
NCHP = NS * (CPT0 + CPT1)
EPAD = NCHP * CHUNK - E
NP = 10240
RPT = NP // NS
GJ = 16
L = 16


def _sc_body(src_hbm, dst_hbm, h_hbm, zagg_hbm, zdeg_hbm,
             agg_out, deg_out,
             agg_sh, srcg, dstg, rows0, rows1, deg_local,
             sem0, sem1):
    c = lax.axis_index("c")
    s = lax.axis_index("s")
    wid = c * NS + s
    start = jnp.where(c == 0, s * CPT0, NS * CPT0 + s * CPT1)
    ngroups = jnp.where(c == 0, CPT0 // GJ, CPT1 // GJ)
    row0 = s * RPT

    pltpu.sync_copy(zagg_hbm, rows0)
    pltpu.sync_copy(zagg_hbm, rows1)
    for r in range(RPT // CHUNK):
        buf = rows0 if r % 2 == 0 else rows1
        pltpu.sync_copy(buf, agg_sh.at[pl.ds(row0 + r * CHUNK, CHUNK)])
    pltpu.sync_copy(zdeg_hbm, deg_local)

    plsc.subcore_barrier()

    ones = jnp.ones((L,), jnp.float32)

    def _deg_add(jrow):
        for k in range(CHUNK // L):
            idx = dstg[jrow, pl.ds(k * L, L)]
            plsc.addupdate_scatter(deg_local, [idx], ones)

    def _group(g, _):
        pltpu.sync_copy(src_hbm.at[pl.ds(start + g * GJ, GJ)], srcg)
        pltpu.sync_copy(dst_hbm.at[pl.ds(start + g * GJ, GJ)], dstg)

        def _pair(p, _):
            j0 = 2 * p
            j1 = j0 + 1
            d0 = pltpu.async_copy(h_hbm.at[srcg.at[j0]], rows0, sem0)
            d1 = pltpu.async_copy(h_hbm.at[srcg.at[j1]], rows1, sem1)
            d0.wait()
            pltpu.sync_copy(rows0, agg_sh.at[dstg.at[j0]], add=True)
            _deg_add(j0)
            d1.wait()
            pltpu.sync_copy(rows1, agg_sh.at[dstg.at[j1]], add=True)
            _deg_add(j1)
            return 0

        return lax.fori_loop(0, GJ // 2, _pair, 0)

    lax.fori_loop(0, ngroups, _group, 0)

    plsc.subcore_barrier()

    for r in range(RPT // CHUNK):
        buf = rows0 if r % 2 == 0 else rows1
        pltpu.sync_copy(agg_sh.at[pl.ds(row0 + r * CHUNK, CHUNK)], buf)
        pltpu.sync_copy(buf, agg_out.at[c, pl.ds(row0 + r * CHUNK, CHUNK)])
    pltpu.sync_copy(deg_local, deg_out.at[wid])


@functools.cache
def _sc_agg():
  return pl.kernel(
    _sc_body,
    out_type=(
        jax.ShapeDtypeStruct((NC, NP, D), jnp.float32),
        jax.ShapeDtypeStruct((NW, NP), jnp.float32),
    ),
    mesh=plsc.VectorSubcoreMesh(core_axis_name="c", subcore_axis_name="s",
                                num_cores=NC, num_subcores=NS),
    compiler_params=pltpu.CompilerParams(needs_layout_passes=False),
    scratch_types=[
        pltpu.VMEM_SHARED((NP, D), jnp.float32),
        pltpu.VMEM((GJ, CHUNK), jnp.int32),
        pltpu.VMEM((GJ, CHUNK), jnp.int32),
        pltpu.VMEM((CHUNK, D), jnp.float32),
        pltpu.VMEM((CHUNK, D), jnp.float32),
        pltpu.VMEM((NP,), jnp.float32),
        pltpu.SemaphoreType.DMA,
        pltpu.SemaphoreType.DMA,
    ],
  )


def _tc_body(h_ref, agg_ref, deg_ref, ws_ref, wn_ref, b_ref, o_ref):
    h = h_ref[...]
    agg = agg_ref[0] + agg_ref[1]
    deg = jnp.sum(deg_ref[...], axis=0)[:, None]
    hn = agg / jnp.maximum(deg, 1.0)
    acc = jnp.dot(h, ws_ref[...], preferred_element_type=jnp.float32)
    acc = acc + jnp.dot(hn, wn_ref[...], preferred_element_type=jnp.float32)
    acc = acc + b_ref[...]
    o_ref[...] = h + jnp.maximum(acc, 0.0)


_TC_ROWS = 1024


def _tc_combine(h_pad, agg2, deg32, W_self, W_neigh, b2d):
    grid = (NP // _TC_ROWS,)
    return pl.pallas_call(
        _tc_body,
        grid=grid,
        in_specs=[
            pl.BlockSpec((_TC_ROWS, D), lambda i: (i, 0)),
            pl.BlockSpec((NC, _TC_ROWS, D), lambda i: (0, i, 0)),
            pl.BlockSpec((NW, _TC_ROWS), lambda i: (0, i)),
            pl.BlockSpec((D, D), lambda i: (0, 0)),
            pl.BlockSpec((D, D), lambda i: (0, 0)),
            pl.BlockSpec((1, D), lambda i: (0, 0)),
        ],
        out_specs=pl.BlockSpec((_TC_ROWS, D), lambda i: (i, 0)),
        out_shape=jax.ShapeDtypeStruct((NP, D), jnp.float32),
    )(h_pad, agg2, deg32, W_self, W_neigh, b2d)


@jax.jit
def kernel(h, edge_index, W_self, W_neigh, b):
    src = edge_index[0].astype(jnp.int32)
    dst = edge_index[1].astype(jnp.int32)
    pad_i = jnp.arange(EPAD, dtype=jnp.int32)
    src2d = jnp.concatenate(
        [src, pad_i % N]).reshape(NCHP, CHUNK)
    dst2d = jnp.concatenate(
        [dst, N + pad_i % (NP - N)]).reshape(NCHP, CHUNK)
    zagg = jnp.zeros((CHUNK, D), jnp.float32)
    zdeg = jnp.zeros((NP,), jnp.float32)
    agg2, deg32 = _sc_agg()(src2d, dst2d, h, zagg, zdeg)
    h_pad = jnp.concatenate([h, jnp.zeros((NP - N, D), jnp.float32)])
    out = _tc_combine(h_pad, agg2, deg32, W_self, W_neigh, b.reshape(1, D))
    return out[:N]


# speedup vs baseline: 3.9674x; 1.0258x over previous
import functools

import jax
import jax.numpy as jnp
from jax import lax
from jax.experimental import pallas as pl
from jax.experimental.pallas import tpu as pltpu
from jax.experimental.pallas import tpu_sc as plsc

N = 10000
E = 320000
D = 128

NC = 2
NS = 16
NW = NC * NS
CHUNK = 128
CPT0 = 80
CPT1 = 80
NCHP = NS * (CPT0 + CPT1)
EPAD = NCHP * CHUNK - E
NP = 10240
RPT = NP // NS
GJ = 16
L = 16


def _sc_body(src_hbm, dst_hbm, h_hbm, zagg_hbm, zdeg_hbm,
             agg_out, deg_out,
             agg_sh, srcg, dstg, rows0, rows1, deg_local,
             sem0, sem1, sem2, sem3):
    c = lax.axis_index("c")
    s = lax.axis_index("s")
    wid = c * NS + s
    start = jnp.where(c == 0, s * CPT0, NS * CPT0 + s * CPT1)
    ngroups = jnp.where(c == 0, CPT0 // GJ, CPT1 // GJ)
    row0 = s * RPT

    pltpu.sync_copy(zagg_hbm, rows0)
    pltpu.sync_copy(zagg_hbm, rows1)
    for r in range(RPT // CHUNK):
        buf = rows0 if r % 2 == 0 else rows1
        pltpu.sync_copy(buf, agg_sh.at[pl.ds(row0 + r * CHUNK, CHUNK)])
    pltpu.sync_copy(zdeg_hbm, deg_local)

    plsc.subcore_barrier()

    ones = jnp.ones((L,), jnp.float32)

    def _deg_add(jrow):
        for k in range(CHUNK // L):
            idx = dstg[jrow, pl.ds(k * L, L)]
            plsc.addupdate_scatter(deg_local, [idx], ones)

    def _group(g, _):
        pltpu.sync_copy(src_hbm.at[pl.ds(start + g * GJ, GJ)], srcg)
        pltpu.sync_copy(dst_hbm.at[pl.ds(start + g * GJ, GJ)], dstg)

        def _pair(p, _):
            j0 = 2 * p
            j1 = j0 + 1
            d0 = pltpu.async_copy(h_hbm.at[srcg.at[j0]], rows0, sem0)
            d1 = pltpu.async_copy(h_hbm.at[srcg.at[j1]], rows1, sem1)
            d0.wait()
            a0 = pltpu.async_copy(rows0, agg_sh.at[dstg.at[j0]], sem2,
                                  add=True)
            _deg_add(j0)
            d1.wait()
            a1 = pltpu.async_copy(rows1, agg_sh.at[dstg.at[j1]], sem3,
                                  add=True)
            _deg_add(j1)
            a0.wait()
            a1.wait()
            return 0

        return lax.fori_loop(0, GJ // 2, _pair, 0)

    lax.fori_loop(0, ngroups, _group, 0)

    plsc.subcore_barrier()

    for r in range(RPT // CHUNK):
        buf = rows0 if r % 2 == 0 else rows1
        pltpu.sync_copy(agg_sh.at[pl.ds(row0 + r * CHUNK, CHUNK)], buf)
        pltpu.sync_copy(buf, agg_out.at[c, pl.ds(row0 + r * CHUNK, CHUNK)])
    pltpu.sync_copy(deg_local, deg_out.at[wid])


@functools.cache
def _sc_agg():
  return pl.kernel(
    _sc_body,
    out_type=(
        jax.ShapeDtypeStruct((NC, NP, D), jnp.float32),
        jax.ShapeDtypeStruct((NW, NP), jnp.float32),
    ),
    mesh=plsc.VectorSubcoreMesh(core_axis_name="c", subcore_axis_name="s",
                                num_cores=NC, num_subcores=NS),
    compiler_params=pltpu.CompilerParams(needs_layout_passes=False),
    scratch_types=[
        pltpu.VMEM_SHARED((NP, D), jnp.float32),
        pltpu.VMEM((GJ, CHUNK), jnp.int32),
        pltpu.VMEM((GJ, CHUNK), jnp.int32),
        pltpu.VMEM((CHUNK, D), jnp.float32),
        pltpu.VMEM((CHUNK, D), jnp.float32),
        pltpu.VMEM((NP,), jnp.float32),
        pltpu.SemaphoreType.DMA,
        pltpu.SemaphoreType.DMA,
        pltpu.SemaphoreType.DMA,
        pltpu.SemaphoreType.DMA,
    ],
  )


def _tc_body(h_ref, agg_ref, deg_ref, ws_ref, wn_ref, b_ref, o_ref):
    h = h_ref[...]
    agg = agg_ref[0] + agg_ref[1]
    deg = jnp.sum(deg_ref[...], axis=0)[:, None]
    hn = agg / jnp.maximum(deg, 1.0)
    acc = jnp.dot(h, ws_ref[...], preferred_element_type=jnp.float32)
    acc = acc + jnp.dot(hn, wn_ref[...], preferred_element_type=jnp.float32)
    acc = acc + b_ref[...]
    o_ref[...] = h + jnp.maximum(acc, 0.0)


_TC_ROWS = 1024


def _tc_combine(h_pad, agg2, deg32, W_self, W_neigh, b2d):
    grid = (NP // _TC_ROWS,)
    return pl.pallas_call(
        _tc_body,
        grid=grid,
        in_specs=[
            pl.BlockSpec((_TC_ROWS, D), lambda i: (i, 0)),
            pl.BlockSpec((NC, _TC_ROWS, D), lambda i: (0, i, 0)),
            pl.BlockSpec((NW, _TC_ROWS), lambda i: (0, i)),
            pl.BlockSpec((D, D), lambda i: (0, 0)),
            pl.BlockSpec((D, D), lambda i: (0, 0)),
            pl.BlockSpec((1, D), lambda i: (0, 0)),
        ],
        out_specs=pl.BlockSpec((_TC_ROWS, D), lambda i: (i, 0)),
        out_shape=jax.ShapeDtypeStruct((NP, D), jnp.float32),
    )(h_pad, agg2, deg32, W_self, W_neigh, b2d)


@jax.jit
def kernel(h, edge_index, W_self, W_neigh, b):
    src = edge_index[0].astype(jnp.int32)
    dst = edge_index[1].astype(jnp.int32)
    pad_i = jnp.arange(EPAD, dtype=jnp.int32)
    src2d = jnp.concatenate(
        [src, pad_i % N]).reshape(NCHP, CHUNK)
    dst2d = jnp.concatenate(
        [dst, N + pad_i % (NP - N)]).reshape(NCHP, CHUNK)
    zagg = jnp.zeros((CHUNK, D), jnp.float32)
    zdeg = jnp.zeros((NP,), jnp.float32)
    agg2, deg32 = _sc_agg()(src2d, dst2d, h, zagg, zdeg)
    h_pad = jnp.concatenate([h, jnp.zeros((NP - N, D), jnp.float32)])
    out = _tc_combine(h_pad, agg2, deg32, W_self, W_neigh, b.reshape(1, D))
    return out[:N]
